# baseline (speedup 1.0000x reference)
import functools

import jax
import jax.numpy as jnp
from jax import lax
from jax.experimental import pallas as pl
from jax.experimental.pallas import tpu as pltpu
from jax.experimental.pallas import tpu_sc as plsc

_B = 16
_SEQ = 2048
_HALF = 1024
_D = 128
_ROWS = _B * _SEQ
_NC = 2
_NS = 16
_NW = _NC * _NS
_PER_W = _ROWS // _NW
_CHUNK = 128
_NCHUNK = _PER_W // _CHUNK


def _combined_body(pos_ref, seg_ref, out_ref):
    i = lax.broadcasted_iota(jnp.int32, (_SEQ, 1), 0)
    seg = jnp.where(i >= _HALF, seg_ref[1:2, :], seg_ref[0:1, :])
    out_ref[:, :] = pos_ref[:, :] + seg


def _make_combined(pos_table, seg_table):
    return pl.pallas_call(
        _combined_body,
        out_shape=jax.ShapeDtypeStruct((_SEQ, _D), jnp.float32),
    )(pos_table, seg_table)


_sc_mesh = plsc.VectorSubcoreMesh(core_axis_name="c", subcore_axis_name="s")


@functools.partial(
    pl.kernel,
    out_type=jax.ShapeDtypeStruct((_ROWS, _D), jnp.float32),
    mesh=_sc_mesh,
    scratch_types=[
        pltpu.VMEM((_CHUNK,), jnp.int32),
        pltpu.VMEM((_CHUNK, _D), jnp.float32),
        pltpu.SemaphoreType.DMA,
    ],
)
def _sc_gather(x_hbm, comb_hbm, tok_hbm, out_hbm, idx_v, rows_v, sem):
    wid = lax.axis_index("s") * _NC + lax.axis_index("c")
    base = wid * _PER_W
    pos0 = (wid % 2) * _HALF
    for j in range(_NCHUNK):
        off = j * _CHUNK
        pltpu.sync_copy(x_hbm.at[pl.ds(base + off, _CHUNK)], idx_v)
        pltpu.sync_copy(comb_hbm.at[pl.ds(pos0 + off, _CHUNK), :], rows_v)
        pltpu.async_copy(tok_hbm.at[idx_v], rows_v, sem, add=True).wait()
        pltpu.sync_copy(rows_v, out_hbm.at[pl.ds(base + off, _CHUNK), :])


def kernel(x, token_table, pos_table, seg_table):
    combined = _make_combined(pos_table, seg_table)
    x_flat = x.reshape(-1).astype(jnp.int32)
    out = _sc_gather(x_flat, combined, token_table)
    return out.reshape(_B, _SEQ, _D)


# speedup vs baseline: 1.1945x; 1.1945x over previous
import functools

import jax
import jax.numpy as jnp
from jax import lax
from jax.experimental import pallas as pl
from jax.experimental.pallas import tpu as pltpu
from jax.experimental.pallas import tpu_sc as plsc

_B = 16
_SEQ = 2048
_HALF = 1024
_D = 128
_ROWS = _B * _SEQ
_NC = 2
_NS = 16
_NW = _NC * _NS
_PER_W = _ROWS // _NW
_CHUNK = 128
_NCHUNK = _PER_W // _CHUNK


def _combined_body(pos_ref, seg_ref, out_ref):
    i = lax.broadcasted_iota(jnp.int32, (_SEQ, 1), 0)
    seg = jnp.where(i >= _HALF, seg_ref[1:2, :], seg_ref[0:1, :])
    out_ref[:, :] = pos_ref[:, :] + seg


def _make_combined(pos_table, seg_table):
    return pl.pallas_call(
        _combined_body,
        out_shape=jax.ShapeDtypeStruct((_SEQ, _D), jnp.float32),
    )(pos_table, seg_table)


_sc_mesh = plsc.VectorSubcoreMesh(core_axis_name="c", subcore_axis_name="s")


@functools.partial(
    pl.kernel,
    out_type=jax.ShapeDtypeStruct((_ROWS, _D), jnp.float32),
    mesh=_sc_mesh,
    scratch_types=[
        pltpu.VMEM((_NCHUNK, _CHUNK), jnp.int32),
        pltpu.VMEM((_CHUNK, _D), jnp.float32),
        pltpu.VMEM((_CHUNK, _D), jnp.float32),
        pltpu.SemaphoreType.DMA,
        pltpu.SemaphoreType.DMA,
        pltpu.SemaphoreType.DMA,
        pltpu.SemaphoreType.DMA,
        pltpu.SemaphoreType.DMA,
    ],
)
def _sc_gather(x_hbm, comb_hbm, tok_hbm, out_hbm,
               idx_v, rows0, rows1, sem_c0, sem_c1, sem_g, sem_o0, sem_o1):
    wid = lax.axis_index("s") * _NC + lax.axis_index("c")
    base = wid * _PER_W
    pos0 = (wid % 2) * _HALF
    rows = (rows0, rows1)
    sem_c = (sem_c0, sem_c1)
    sem_o = (sem_o0, sem_o1)

    pltpu.sync_copy(x_hbm.at[pl.ds(wid * _NCHUNK, _NCHUNK), :], idx_v)

    comb_cp = [None, None]
    out_cp = [None, None]
    comb_cp[0] = pltpu.async_copy(
        comb_hbm.at[pl.ds(pos0, _CHUNK), :], rows[0], sem_c[0])
    for j in range(_NCHUNK):
        p = j % 2
        q = 1 - p
        if j + 1 < _NCHUNK:
            if j >= 1:
                out_cp[q].wait()
            comb_cp[q] = pltpu.async_copy(
                comb_hbm.at[pl.ds(pos0 + (j + 1) * _CHUNK, _CHUNK), :],
                rows[q], sem_c[q])
        comb_cp[p].wait()
        pltpu.async_copy(tok_hbm.at[idx_v.at[j]], rows[p], sem_g,
                         add=True).wait()
        out_cp[p] = pltpu.async_copy(
            rows[p], out_hbm.at[pl.ds(base + j * _CHUNK, _CHUNK), :], sem_o[p])
    out_cp[0].wait()
    out_cp[1].wait()


def kernel(x, token_table, pos_table, seg_table):
    combined = _make_combined(pos_table, seg_table)
    x2d = x.reshape(_ROWS // _CHUNK, _CHUNK).astype(jnp.int32)
    out = _sc_gather(x2d, combined, token_table)
    return out.reshape(_B, _SEQ, _D)


# speedup vs baseline: 1.3926x; 1.1658x over previous
import functools

import jax
import jax.numpy as jnp
from jax import lax
from jax.experimental import pallas as pl
from jax.experimental.pallas import tpu as pltpu
from jax.experimental.pallas import tpu_sc as plsc

_B = 16
_SEQ = 2048
_HALF = 1024
_D = 128
_ROWS = _B * _SEQ
_NC = 2
_NS = 16
_NW = _NC * _NS
_PER_W = _ROWS // _NW
_CHUNK = 128
_NCHUNK = _PER_W // _CHUNK


def _combined_body(pos_ref, seg_ref, out_ref):
    i = lax.broadcasted_iota(jnp.int32, (_SEQ, 1), 0)
    seg = jnp.where(i >= _HALF, seg_ref[1:2, :], seg_ref[0:1, :])
    out_ref[:, :] = pos_ref[:, :] + seg


def _make_combined(pos_table, seg_table):
    return pl.pallas_call(
        _combined_body,
        out_shape=jax.ShapeDtypeStruct((_SEQ, _D), jnp.float32),
    )(pos_table, seg_table)


_sc_mesh = plsc.VectorSubcoreMesh(core_axis_name="c", subcore_axis_name="s")


@functools.partial(
    pl.kernel,
    out_type=jax.ShapeDtypeStruct((_ROWS, _D), jnp.float32),
    mesh=_sc_mesh,
    scratch_types=[
        pltpu.VMEM((_NCHUNK, _CHUNK), jnp.int32),
        pltpu.VMEM((_CHUNK, _D), jnp.float32),
        pltpu.VMEM((_CHUNK, _D), jnp.float32),
        pltpu.VMEM_SHARED((_SEQ, _D), jnp.float32),
        pltpu.SemaphoreType.DMA,
        pltpu.SemaphoreType.DMA,
        pltpu.SemaphoreType.DMA,
        pltpu.SemaphoreType.DMA,
        pltpu.SemaphoreType.DMA,
    ],
)
def _sc_gather(x_hbm, comb_hbm, tok_hbm, out_hbm,
               idx_v, rows0, rows1, comb_sp,
               sem_c0, sem_c1, sem_g, sem_o0, sem_o1):
    sid = lax.axis_index("s")
    wid = sid * _NC + lax.axis_index("c")
    base = wid * _PER_W
    pos0 = (wid % 2) * _HALF
    rows = (rows0, rows1)
    sem_c = (sem_c0, sem_c1)
    sem_o = (sem_o0, sem_o1)

    pltpu.sync_copy(comb_hbm.at[pl.ds(sid * _CHUNK, _CHUNK), :],
                    comb_sp.at[pl.ds(sid * _CHUNK, _CHUNK), :])

    pltpu.sync_copy(x_hbm.at[pl.ds(wid * _NCHUNK, _NCHUNK), :], idx_v)
    plsc.subcore_barrier()

    comb_cp = [None, None]
    out_cp = [None, None]
    comb_cp[0] = pltpu.async_copy(
        comb_sp.at[pl.ds(pos0, _CHUNK), :], rows[0], sem_c[0])
    for j in range(_NCHUNK):
        p = j % 2
        q = 1 - p
        if j + 1 < _NCHUNK:
            if j >= 1:
                out_cp[q].wait()
            comb_cp[q] = pltpu.async_copy(
                comb_sp.at[pl.ds(pos0 + (j + 1) * _CHUNK, _CHUNK), :],
                rows[q], sem_c[q])
        comb_cp[p].wait()
        pltpu.async_copy(tok_hbm.at[idx_v.at[j]], rows[p], sem_g,
                         add=True).wait()
        out_cp[p] = pltpu.async_copy(
            rows[p], out_hbm.at[pl.ds(base + j * _CHUNK, _CHUNK), :], sem_o[p])
    out_cp[0].wait()
    out_cp[1].wait()


def kernel(x, token_table, pos_table, seg_table):
    combined = _make_combined(pos_table, seg_table)
    x2d = x.reshape(_ROWS // _CHUNK, _CHUNK).astype(jnp.int32)
    out = _sc_gather(x2d, combined, token_table)
    return out.reshape(_B, _SEQ, _D)


# speedup vs baseline: 1.5395x; 1.1055x over previous
import functools

import jax
import jax.numpy as jnp
from jax import lax
from jax.experimental import pallas as pl
from jax.experimental.pallas import tpu as pltpu
from jax.experimental.pallas import tpu_sc as plsc

_B = 16
_SEQ = 2048
_HALF = 1024
_D = 128
_ROWS = _B * _SEQ
_NC = 2
_NS = 16
_NW = _NC * _NS
_PER_W = _ROWS // _NW
_CHUNK = 128
_NCHUNK = _PER_W // _CHUNK


def _combined_body(pos_ref, seg_ref, out_ref):
    i = lax.broadcasted_iota(jnp.int32, (_SEQ, 1), 0)
    seg = jnp.where(i >= _HALF, seg_ref[1:2, :], seg_ref[0:1, :])
    out_ref[:, :] = pos_ref[:, :] + seg


def _make_combined(pos_table, seg_table):
    return pl.pallas_call(
        _combined_body,
        out_shape=jax.ShapeDtypeStruct((_SEQ, _D), jnp.float32),
    )(pos_table, seg_table)


_sc_mesh = plsc.VectorSubcoreMesh(core_axis_name="c", subcore_axis_name="s")


@functools.partial(
    pl.kernel,
    out_type=jax.ShapeDtypeStruct((_ROWS, _D), jnp.float32),
    mesh=_sc_mesh,
    scratch_types=[
        pltpu.VMEM((_NCHUNK, _CHUNK), jnp.int32),
        pltpu.VMEM((_CHUNK, _D), jnp.float32),
        pltpu.VMEM((_CHUNK, _D), jnp.float32),
        pltpu.VMEM((_CHUNK, _D), jnp.float32),
        pltpu.VMEM((_CHUNK, _D), jnp.float32),
        pltpu.VMEM_SHARED((_SEQ, _D), jnp.float32),
        pltpu.SemaphoreType.DMA,
        pltpu.SemaphoreType.DMA,
        pltpu.SemaphoreType.DMA,
        pltpu.SemaphoreType.DMA,
        pltpu.SemaphoreType.DMA,
        pltpu.SemaphoreType.DMA,
        pltpu.SemaphoreType.DMA,
        pltpu.SemaphoreType.DMA,
        pltpu.SemaphoreType.DMA,
        pltpu.SemaphoreType.DMA,
        pltpu.SemaphoreType.DMA,
        pltpu.SemaphoreType.DMA,
    ],
)
def _sc_gather(x_hbm, comb_hbm, tok_hbm, out_hbm,
               idx_v, rows0, rows1, rows2, rows3, comb_sp,
               sc0, sc1, sc2, sc3, sg0, sg1, sg2, sg3, so0, so1, so2, so3):
    sid = lax.axis_index("s")
    wid = sid * _NC + lax.axis_index("c")
    base = wid * _PER_W
    pos0 = (wid % 2) * _HALF
    rows = (rows0, rows1, rows2, rows3)
    sem_c = (sc0, sc1, sc2, sc3)
    sem_g = (sg0, sg1, sg2, sg3)
    sem_o = (so0, so1, so2, so3)

    pltpu.sync_copy(comb_hbm.at[pl.ds(sid * _CHUNK, _CHUNK), :],
                    comb_sp.at[pl.ds(sid * _CHUNK, _CHUNK), :])

    pltpu.sync_copy(x_hbm.at[pl.ds(wid * _NCHUNK, _NCHUNK), :], idx_v)
    plsc.subcore_barrier()

    def comb_load(j):
        return pltpu.async_copy(
            comb_sp.at[pl.ds(pos0 + j * _CHUNK, _CHUNK), :],
            rows[j % 4], sem_c[j % 4])

    def gather(j):
        return pltpu.async_copy(tok_hbm.at[idx_v.at[j]], rows[j % 4],
                                sem_g[j % 4], add=True)

    def out_store(j):
        return pltpu.async_copy(
            rows[j % 4], out_hbm.at[pl.ds(base + j * _CHUNK, _CHUNK), :],
            sem_o[j % 4])

    cps = {}
    for j in range(3):
        cps["c", j] = comb_load(j)
    for j in range(_NCHUNK):
        cps["c", j].wait()
        cps["g", j] = gather(j)
        if j >= 1:
            cps["g", j - 1].wait()
            cps["o", j - 1] = out_store(j - 1)
        if j + 3 < _NCHUNK:
            if j >= 1:
                cps["o", j - 1].wait()
            cps["c", j + 3] = comb_load(j + 3)
    cps["g", _NCHUNK - 1].wait()
    cps["o", _NCHUNK - 1] = out_store(_NCHUNK - 1)
    for j in range(4, _NCHUNK):
        cps["o", j].wait()


def kernel(x, token_table, pos_table, seg_table):
    combined = _make_combined(pos_table, seg_table)
    x2d = x.reshape(_ROWS // _CHUNK, _CHUNK).astype(jnp.int32)
    out = _sc_gather(x2d, combined, token_table)
    return out.reshape(_B, _SEQ, _D)
